# baseline (speedup 1.0000x reference)
import functools

import jax
import jax.numpy as jnp
from jax import lax
from jax.experimental import pallas as pl
from jax.experimental.pallas import tpu as pltpu
from jax.experimental.pallas import tpu_sc as plsc

N_NODES = 10000
D = 128
NP = 10240
NC = 2
NS = 16
NW = NC * NS
CHUNK = 128
RPT = NP // NS
L = 16


def _mesh():
    return plsc.VectorSubcoreMesh(core_axis_name="c", subcore_axis_name="s")


def _make_hist(ep):
    kch = ep // (NW * CHUNK)
    fire = 8

    @functools.partial(
        pl.kernel,
        mesh=_mesh(),
        out_type=jax.ShapeDtypeStruct((NC * NP,), jnp.float32),
        scratch_types=[
            pltpu.VMEM((kch, CHUNK), jnp.int32),
            pltpu.VMEM((CHUNK,), jnp.float32),
            pltpu.VMEM_SHARED((NP,), jnp.float32),
            pltpu.SemaphoreType.DMA,
        ],
    )
    def hist(dstr_hbm, zeros_hbm, out_hbm, di_all, ones_v, acc_sh, sem):
        c = lax.axis_index("c")
        s = lax.axis_index("s")
        wid = s * NC + c
        for j in range(CHUNK // L):
            ones_v[pl.ds(j * L, L)] = jnp.full((L,), 1.0, jnp.float32)
        r0 = s * RPT
        pltpu.sync_copy(zeros_hbm.at[pl.ds(0, RPT)], acc_sh.at[pl.ds(r0, RPT)])
        pltpu.sync_copy(dstr_hbm.at[pl.ds(wid * kch, kch)], di_all)
        plsc.subcore_barrier()

        def body(t, carry):
            for j in range(fire):
                pltpu.async_copy(ones_v, acc_sh.at[di_all.at[t * fire + j]],
                                 sem, add=True)
            for j in range(fire):
                pltpu.make_async_copy(ones_v, acc_sh.at[di_all.at[0]],
                                      sem).wait()
            return carry

        lax.fori_loop(0, kch // fire, body, 0)
        plsc.subcore_barrier()
        pltpu.sync_copy(acc_sh.at[pl.ds(r0, RPT)],
                        out_hbm.at[pl.ds(c * NP + r0, RPT)])

    return hist


def _make_scatter(ep):
    kch = ep // (NW * CHUNK)

    @functools.partial(
        pl.kernel,
        mesh=_mesh(),
        out_type=jax.ShapeDtypeStruct((NC * NP, D), jnp.float32),
        scratch_types=[
            pltpu.VMEM((CHUNK,), jnp.int32),
            pltpu.VMEM((CHUNK,), jnp.int32),
            pltpu.VMEM((CHUNK, D), jnp.float32),
            pltpu.VMEM_SHARED((NP, D), jnp.float32),
            pltpu.SemaphoreType.DMA,
        ],
    )
    def scat(src_hbm, dst_hbm, g_hbm, zeros_hbm, out_hbm,
             si_v, di_v, rows_v, acc_sh, sem):
        c = lax.axis_index("c")
        s = lax.axis_index("s")
        wid = s * NC + c
        r0 = s * RPT
        pltpu.sync_copy(zeros_hbm, acc_sh.at[pl.ds(r0, RPT)])
        plsc.subcore_barrier()
        base = wid * (kch * CHUNK)

        def body(gi, carry):
            e0 = base + gi * CHUNK
            pltpu.sync_copy(src_hbm.at[pl.ds(e0, CHUNK)], si_v)
            pltpu.sync_copy(dst_hbm.at[pl.ds(e0, CHUNK)], di_v)
            pltpu.async_copy(g_hbm.at[si_v], rows_v, sem).wait()
            pltpu.sync_copy(rows_v, acc_sh.at[di_v], add=True)
            return carry

        lax.fori_loop(0, kch, body, 0)
        plsc.subcore_barrier()
        pltpu.sync_copy(acc_sh.at[pl.ds(r0, RPT)],
                        out_hbm.at[pl.ds(c * NP + r0, RPT)])

    return scat


BLK = 1000


def _mm_body(x_ref, w_ref, h0_ref, h1_ref, g_ref, dinv_ref):
    deg = 1.0 + h0_ref[...] + h1_ref[...]
    dinv = lax.rsqrt(deg)
    h = jnp.dot(x_ref[...], w_ref[...], preferred_element_type=jnp.float32)
    g_ref[...] = h * dinv
    dinv_ref[...] = dinv


def _mm_call(x, W, h0, h1):
    grid = N_NODES // BLK
    return pl.pallas_call(
        _mm_body,
        grid=(grid,),
        in_specs=[
            pl.BlockSpec((BLK, D), lambda i: (i, 0)),
            pl.BlockSpec((D, D), lambda i: (0, 0)),
            pl.BlockSpec((BLK, 1), lambda i: (i, 0)),
            pl.BlockSpec((BLK, 1), lambda i: (i, 0)),
        ],
        out_specs=[
            pl.BlockSpec((BLK, D), lambda i: (i, 0)),
            pl.BlockSpec((BLK, 1), lambda i: (i, 0)),
        ],
        out_shape=[
            jax.ShapeDtypeStruct((N_NODES, D), jnp.float32),
            jax.ShapeDtypeStruct((N_NODES, 1), jnp.float32),
        ],
    )(x, W, h0, h1)


BLK4 = 640


def _ep_body(a0_ref, a1_ref, g_ref, dinv_ref, b_ref, o_ref):
    a = a0_ref[...] + a1_ref[...] + g_ref[...]
    o_ref[...] = jnp.maximum(a * dinv_ref[...] + b_ref[...], 0.0)


def _ep_call(acc, g, dinv, b2):
    grid = (N_NODES + BLK4 - 1) // BLK4
    return pl.pallas_call(
        _ep_body,
        grid=(grid,),
        in_specs=[
            pl.BlockSpec((BLK4, D), lambda i: (i, 0)),
            pl.BlockSpec((BLK4, D), lambda i: (i + NP // BLK4, 0)),
            pl.BlockSpec((BLK4, D), lambda i: (i, 0)),
            pl.BlockSpec((BLK4, 1), lambda i: (i, 0)),
            pl.BlockSpec((1, D), lambda i: (0, 0)),
        ],
        out_specs=pl.BlockSpec((BLK4, D), lambda i: (i, 0)),
        out_shape=jax.ShapeDtypeStruct((N_NODES, D), jnp.float32),
    )(acc, acc, g, dinv, b2)


def kernel(x, edge_index, W, b):
    src = edge_index[0]
    dst = edge_index[1]
    e = src.shape[0]
    kch = -(-e // (NW * CHUNK))
    kch = ((kch + 7) // 8) * 8
    ep = kch * NW * CHUNK
    pad = ep - e
    srcp = jnp.concatenate([src, jnp.zeros((pad,), jnp.int32)])
    dstp = jnp.concatenate([dst, jnp.full((pad,), N_NODES, jnp.int32)])
    dstr = dstp.reshape(ep // CHUNK, CHUNK)

    zeros1 = jnp.zeros((RPT,), jnp.float32)
    zeros2 = jnp.zeros((RPT, D), jnp.float32)

    hist = _make_hist(ep)(dstr, zeros1)
    h0 = hist[:NP].reshape(NP, 1)[:N_NODES]
    h1 = hist[NP:].reshape(NP, 1)[:N_NODES]

    g, dinv = _mm_call(x, W, h0, h1)

    acc = _make_scatter(ep)(srcp, dstp, g, zeros2)

    b2 = b.reshape(1, D)
    return _ep_call(acc, g, dinv, b2)


# speedup vs baseline: 1.2847x; 1.2847x over previous
import functools

import jax
import jax.numpy as jnp
from jax import lax
from jax.experimental import pallas as pl
from jax.experimental.pallas import tpu as pltpu
from jax.experimental.pallas import tpu_sc as plsc

N_NODES = 10000
D = 128
NP = 10240
NC = 2
NS = 16
NW = NC * NS
CHUNK = 128
RPT = NP // NS


def _mesh():
    return plsc.VectorSubcoreMesh(core_axis_name="c", subcore_axis_name="s")


def _make_hist(ep):
    chunks = ep // (NW * CHUNK)
    eper = ep // NW

    @functools.partial(
        pl.kernel,
        mesh=_mesh(),
        out_type=jax.ShapeDtypeStruct((NC * NP,), jnp.float32),
        scratch_types=[
            pltpu.VMEM((CHUNK,), jnp.int32),
            pltpu.VMEM((CHUNK,), jnp.float32),
            pltpu.VMEM_SHARED((NP,), jnp.float32),
        ],
    )
    def hist(dst_hbm, zeros_hbm, out_hbm, idx_v, ones_v, acc_sh):
        c = lax.axis_index("c")
        s = lax.axis_index("s")
        wid = s * NC + c
        for j in range(CHUNK // 16):
            ones_v[pl.ds(j * 16, 16)] = jnp.full((16,), 1.0, jnp.float32)
        r0 = s * RPT
        pltpu.sync_copy(zeros_hbm.at[pl.ds(0, RPT)], acc_sh.at[pl.ds(r0, RPT)])
        plsc.subcore_barrier()
        base = wid * eper

        def body(gi, carry):
            pltpu.sync_copy(dst_hbm.at[pl.ds(base + gi * CHUNK, CHUNK)], idx_v)
            pltpu.sync_copy(ones_v, acc_sh.at[idx_v], add=True)
            return carry

        lax.fori_loop(0, chunks, body, 0)
        plsc.subcore_barrier()
        pltpu.sync_copy(acc_sh.at[pl.ds(r0, RPT)],
                        out_hbm.at[pl.ds(c * NP + r0, RPT)])

    return hist


def _make_scatter(ep):
    chunks = ep // (NW * CHUNK)
    eper = ep // NW

    @functools.partial(
        pl.kernel,
        mesh=_mesh(),
        out_type=jax.ShapeDtypeStruct((NC * NP, D), jnp.float32),
        scratch_types=[
            pltpu.VMEM((CHUNK,), jnp.int32),
            pltpu.VMEM((CHUNK,), jnp.int32),
            pltpu.VMEM((CHUNK, D), jnp.float32),
            pltpu.VMEM_SHARED((NP, D), jnp.float32),
            pltpu.SemaphoreType.DMA,
        ],
    )
    def scat(src_hbm, dst_hbm, g_hbm, zeros_hbm, out_hbm,
             si_v, di_v, rows_v, acc_sh, sem):
        c = lax.axis_index("c")
        s = lax.axis_index("s")
        wid = s * NC + c
        r0 = s * RPT
        pltpu.sync_copy(zeros_hbm, acc_sh.at[pl.ds(r0, RPT)])
        plsc.subcore_barrier()
        base = wid * eper

        def body(gi, carry):
            e0 = base + gi * CHUNK
            pltpu.sync_copy(src_hbm.at[pl.ds(e0, CHUNK)], si_v)
            pltpu.sync_copy(dst_hbm.at[pl.ds(e0, CHUNK)], di_v)
            pltpu.async_copy(g_hbm.at[si_v], rows_v, sem).wait()
            pltpu.sync_copy(rows_v, acc_sh.at[di_v], add=True)
            return carry

        lax.fori_loop(0, chunks, body, 0)
        plsc.subcore_barrier()
        pltpu.sync_copy(acc_sh.at[pl.ds(r0, RPT)],
                        out_hbm.at[pl.ds(c * NP + r0, RPT)])

    return scat


BLK = 1000


def _mm_body(x_ref, w_ref, h0_ref, h1_ref, g_ref, dinv_ref):
    deg = 1.0 + h0_ref[...] + h1_ref[...]
    dinv = lax.rsqrt(deg)
    h = jnp.dot(x_ref[...], w_ref[...], preferred_element_type=jnp.float32)
    g_ref[...] = h * dinv
    dinv_ref[...] = dinv


def _mm_call(x, W, h0, h1):
    grid = N_NODES // BLK
    return pl.pallas_call(
        _mm_body,
        grid=(grid,),
        in_specs=[
            pl.BlockSpec((BLK, D), lambda i: (i, 0)),
            pl.BlockSpec((D, D), lambda i: (0, 0)),
            pl.BlockSpec((BLK, 1), lambda i: (i, 0)),
            pl.BlockSpec((BLK, 1), lambda i: (i, 0)),
        ],
        out_specs=[
            pl.BlockSpec((BLK, D), lambda i: (i, 0)),
            pl.BlockSpec((BLK, 1), lambda i: (i, 0)),
        ],
        out_shape=[
            jax.ShapeDtypeStruct((N_NODES, D), jnp.float32),
            jax.ShapeDtypeStruct((N_NODES, 1), jnp.float32),
        ],
    )(x, W, h0, h1)


BLK4 = 640


def _ep_body(a0_ref, a1_ref, g_ref, dinv_ref, b_ref, o_ref):
    a = a0_ref[...] + a1_ref[...] + g_ref[...]
    o_ref[...] = jnp.maximum(a * dinv_ref[...] + b_ref[...], 0.0)


def _ep_call(acc, g, dinv, b2):
    grid = (N_NODES + BLK4 - 1) // BLK4
    return pl.pallas_call(
        _ep_body,
        grid=(grid,),
        in_specs=[
            pl.BlockSpec((BLK4, D), lambda i: (i, 0)),
            pl.BlockSpec((BLK4, D), lambda i: (i + NP // BLK4, 0)),
            pl.BlockSpec((BLK4, D), lambda i: (i, 0)),
            pl.BlockSpec((BLK4, 1), lambda i: (i, 0)),
            pl.BlockSpec((1, D), lambda i: (0, 0)),
        ],
        out_specs=pl.BlockSpec((BLK4, D), lambda i: (i, 0)),
        out_shape=jax.ShapeDtypeStruct((N_NODES, D), jnp.float32),
    )(acc, acc, g, dinv, b2)


def kernel(x, edge_index, W, b):
    src = edge_index[0]
    dst = edge_index[1]
    e = src.shape[0]
    ep = ((e + NW * CHUNK - 1) // (NW * CHUNK)) * (NW * CHUNK)
    pad = ep - e
    srcp = jnp.concatenate([src, jnp.zeros((pad,), jnp.int32)])
    dstp = jnp.concatenate([dst, jnp.full((pad,), N_NODES, jnp.int32)])

    zeros1 = jnp.zeros((RPT,), jnp.float32)
    zeros2 = jnp.zeros((RPT, D), jnp.float32)

    hist = _make_hist(ep)(dstp, zeros1)
    h0 = hist[:NP].reshape(NP, 1)[:N_NODES]
    h1 = hist[NP:].reshape(NP, 1)[:N_NODES]

    g, dinv = _mm_call(x, W, h0, h1)

    acc = _make_scatter(ep)(srcp, dstp, g, zeros2)

    b2 = b.reshape(1, D)
    return _ep_call(acc, g, dinv, b2)
